# baseline (speedup 1.0000x reference)
import functools

import jax
import jax.numpy as jnp
import numpy as np
from jax import lax
from jax.experimental import pallas as pl
from jax.experimental.pallas import tpu as pltpu
from jax.experimental.pallas import tpu_sc as plsc

N = 10000
E = 160000
BE = 1280
BN = 2000
CH = 128
NW = 32
GRP = 2 * CH
EP = 163840
NP2 = 16384
GE = EP // (NW * GRP)
GN = NP2 // (NW * GRP)

_f32 = jnp.float32
_HI = jax.lax.Precision.HIGHEST


def _dot(a, b):
    return jnp.dot(a, b, precision=_HI, preferred_element_type=_f32)


def _b1dot(a, b):
    return jnp.dot(a.astype(jnp.bfloat16), b.astype(jnp.bfloat16),
                   preferred_element_type=_f32)


def _bdot(a, b):
    return _b1dot(a, b)


def _node_enc_body(xn, nt, wsm, cb, fwa, fwb, fb, a1, b1w, n1a, p1, p2, p3):
    mc = _dot(xn[...], wsm[...]) + cb[...]
    h = _bdot(mc, fwa[...]) + _bdot(nt[...], fwb[...]) + fb[...]
    p1[...] = _bdot(h, a1[...])
    p2[...] = _bdot(h, b1w[...])
    p3[...] = _bdot(h, n1a[...])


def _edge1_body(gr, gc, ea, wc, b1, w2, b2, o):
    u = gr[...] + gc[...] + _bdot(ea[...], wc[...]) + b1[...]
    o[...] = _bdot(jax.nn.relu(u), w2[...]) + b2[...]


def _node1_body(g3, ea1n, n1b, n1b1, n1w2, n1b2, e2a, e2b, q1, q2):
    u = jax.nn.relu(g3[...] + _bdot(ea1n[...], n1b[...]) + n1b1[...])
    h1 = _bdot(u, n1w2[...]) + n1b2[...]
    q1[...] = _bdot(h1, e2a[...])
    q2[...] = _bdot(h1, e2b[...])


def _edge2_body(gr2, gc2, ea1, e2c, e2b1, e2w2, e2b2, fw1, fb1, z_ref, st_ref):
    u = jax.nn.relu(gr2[...] + gc2[...] +
                    _bdot(ea1[...], e2c[...]) + e2b1[...])
    v = _bdot(u, e2w2[...]) + e2b2[...]
    z = _bdot(v, fw1[...]) + fb1[...]
    z_ref[...] = z
    s = jnp.sum(z, axis=0).reshape(1, 128)
    sq = jnp.sum(z * z, axis=0).reshape(1, 128)
    upd = jnp.concatenate([s, sq, jnp.zeros((6, 128), _f32)], axis=0)

    @pl.when(pl.program_id(0) == 0)
    def _():
        st_ref[...] = jnp.zeros((8, 128), _f32)

    st_ref[...] += upd


def _final_body(z, av, cv, fw2, fb2, o):
    u = jax.nn.relu(z[...] * av[...] + cv[...])
    o[...] = _bdot(u, fw2[...]) + fb2[...]


def _w_spec(shape):
    return pl.BlockSpec(shape, lambda i: (0,) * len(shape))


def _run_node_enc(Xn, nt, WSM, cb, fwa, fwb, fb, a1, b1w, n1a):
    grid = (N // BN,)
    blk = lambda r, c: pl.BlockSpec((r, c), lambda i: (i, 0))
    return pl.pallas_call(
        _node_enc_body,
        grid=grid,
        in_specs=[blk(BN, 20), blk(BN, 1), _w_spec((20, 128)), _w_spec((1, 128)),
                  _w_spec((128, 128)), _w_spec((1, 128)), _w_spec((1, 128)),
                  _w_spec((128, 128)), _w_spec((128, 128)), _w_spec((128, 128))],
        out_specs=[blk(BN, 128)] * 3,
        out_shape=[jax.ShapeDtypeStruct((N, 128), _f32)] * 3,
    )(Xn, nt, WSM, cb, fwa, fwb, fb, a1, b1w, n1a)


def _run_edge1(gr, gc, ea, wc, b1, w2, b2):
    grid = (E // BE,)
    blk = lambda r, c: pl.BlockSpec((r, c), lambda i: (i, 0))
    return pl.pallas_call(
        _edge1_body,
        grid=grid,
        in_specs=[blk(BE, 128), blk(BE, 128), blk(BE, 4), _w_spec((4, 128)),
                  _w_spec((1, 128)), _w_spec((128, 128)), _w_spec((1, 128))],
        out_specs=blk(BE, 128),
        out_shape=jax.ShapeDtypeStruct((E, 128), _f32),
    )(gr, gc, ea, wc, b1, w2, b2)


def _run_node1(g3p, ea1, n1b, n1b1, n1w2, n1b2, e2a, e2b):
    grid = (N // BN,)
    blk = lambda r, c: pl.BlockSpec((r, c), lambda i: (i, 0))
    return pl.pallas_call(
        _node1_body,
        grid=grid,
        in_specs=[blk(BN, 128), blk(BN, 128), _w_spec((128, 128)), _w_spec((1, 128)),
                  _w_spec((128, 128)), _w_spec((1, 128)),
                  _w_spec((128, 128)), _w_spec((128, 128))],
        out_specs=[blk(BN, 128)] * 2,
        out_shape=[jax.ShapeDtypeStruct((N, 128), _f32)] * 2,
    )(g3p, ea1, n1b, n1b1, n1w2, n1b2, e2a, e2b)


def _run_edge2(gr2, gc2, ea1, e2c, e2b1, e2w2, e2b2, fw1, fb1):
    grid = (E // BE,)
    blk = lambda r, c: pl.BlockSpec((r, c), lambda i: (i, 0))
    return pl.pallas_call(
        _edge2_body,
        grid=grid,
        in_specs=[blk(BE, 128), blk(BE, 128), blk(BE, 128), _w_spec((128, 128)),
                  _w_spec((1, 128)), _w_spec((128, 128)), _w_spec((1, 128)),
                  _w_spec((128, 128)), _w_spec((1, 128))],
        out_specs=[blk(BE, 128), pl.BlockSpec((8, 128), lambda i: (0, 0))],
        out_shape=[jax.ShapeDtypeStruct((E, 128), _f32),
                   jax.ShapeDtypeStruct((8, 128), _f32)],
    )(gr2, gc2, ea1, e2c, e2b1, e2w2, e2b2, fw1, fb1)


def _run_final(z, av, cv, fw2, fb2):
    grid = (E // BE,)
    blk = lambda r, c: pl.BlockSpec((r, c), lambda i: (i, 0))
    return pl.pallas_call(
        _final_body,
        grid=grid,
        in_specs=[blk(BE, 128), _w_spec((1, 128)), _w_spec((1, 128)),
                  _w_spec((128, 3)), _w_spec((1, 3))],
        out_specs=blk(BE, 3),
        out_shape=jax.ShapeDtypeStruct((E, 3), _f32),
    )(z, av, cv, fw2, fb2)


def _sc_mesh():
    return plsc.VectorSubcoreMesh(core_axis_name="c", subcore_axis_name="s")


def _gather_phase(w, tbl, idx_hbm, out_hbm, ngroups, idx_all, bufA, bufB,
                  semA, semB):
    rows_w = ngroups * GRP
    base_w = w * rows_w
    pltpu.sync_copy(idx_hbm.at[pl.ds(base_w, rows_w)],
                    idx_all.at[pl.ds(0, rows_w)])

    def issue(g, buf, sem):
        for sub in range(2):
            pltpu.async_copy(
                tbl.at[idx_all.at[pl.ds(g * GRP + sub * CH, CH)]],
                buf.at[pl.ds(sub * CH, CH)], sem)

    def drain(buf, sem):
        pltpu.make_async_copy(tbl.at[pl.ds(0, GRP)], buf, sem).wait()

    issue(0, bufA, semA)
    issue(1, bufB, semB)

    @pl.loop(0, ngroups, step=2)
    def _(p):
        for off, buf, sem in ((0, bufA, semA), (1, bufB, semB)):
            g = p + off
            drain(buf, sem)
            pltpu.sync_copy(buf, out_hbm.at[pl.ds(base_w + g * GRP, GRP)])

            @pl.when(g + 2 < ngroups)
            def _():
                issue(g + 2, buf, sem)


_SC_SCRATCH = [pltpu.VMEM((GE * GRP,), jnp.int32),
               pltpu.VMEM((GRP, 128), _f32),
               pltpu.VMEM((GRP, 128), _f32),
               pltpu.SemaphoreType.DMA,
               pltpu.SemaphoreType.DMA]


def _sc_gather3(p1, p2, p3, rowE, colE, rowN):
    @functools.partial(
        pl.kernel,
        mesh=_sc_mesh(),
        out_type=(jax.ShapeDtypeStruct((EP, 128), _f32),
                  jax.ShapeDtypeStruct((EP, 128), _f32),
                  jax.ShapeDtypeStruct((NP2, 128), _f32)),
        scratch_types=list(_SC_SCRATCH),
    )
    def k(p1_h, p2_h, p3_h, row_h, col_h, rown_h, gr_h, gc_h, g3_h,
          idx_all, bufA, bufB, semA, semB):
        wid = lax.axis_index("s") * 2 + lax.axis_index("c")
        _gather_phase(wid, p1_h, row_h, gr_h, GE, idx_all, bufA, bufB, semA, semB)
        _gather_phase(wid, p2_h, col_h, gc_h, GE, idx_all, bufA, bufB, semA, semB)
        _gather_phase(wid, p3_h, rown_h, g3_h, GN, idx_all, bufA, bufB, semA, semB)

    return k(p1, p2, p3, rowE, colE, rowN)


def _sc_gather2(q1, q2, rowE, colE):
    @functools.partial(
        pl.kernel,
        mesh=_sc_mesh(),
        out_type=(jax.ShapeDtypeStruct((EP, 128), _f32),
                  jax.ShapeDtypeStruct((EP, 128), _f32)),
        scratch_types=list(_SC_SCRATCH),
    )
    def k(q1_h, q2_h, row_h, col_h, gr_h, gc_h, idx_all, bufA, bufB, semA, semB):
        wid = lax.axis_index("s") * 2 + lax.axis_index("c")
        _gather_phase(wid, q1_h, row_h, gr_h, GE, idx_all, bufA, bufB, semA, semB)
        _gather_phase(wid, q2_h, col_h, gc_h, GE, idx_all, bufA, bufB, semA, semB)

    return k(q1, q2, rowE, colE)


def kernel(x, edge_index, edge_attr, batch, node_type, emb, conv_w, conv_b,
           fuse_w, fuse_b, e1_w1, e1_b1, e1_w2, e1_b2, n1_w1, n1_b1, n1_w2,
           n1_b2, e2_w1, e2_b1, e2_w2, e2_b2, n2_w1, n2_b1, n2_w2, n2_b2,
           f_w1, f_b1, bn_g, bn_b, f_w2, f_b2):
    L = 5
    _b = lambda v: jax.lax.reduce_precision(v, 8, 7)
    cwb = _b(conv_w)
    embb = _b(emb)
    w0 = cwb[:, :, 0]; w1 = cwb[:, :, 1]; w2 = cwb[:, :, 2]
    ws = w0 + w1 + w2
    M = jnp.concatenate([ws[:, :4].T, -w2[:, :4].T, -w0[:, :4].T], axis=0) / L
    s_pe = embb.sum(0)
    const = (s_pe @ ws[:, 4:].T - embb[4] @ w0[:, 4:].T - embb[0] @ w2[:, 4:].T) / L + conv_b
    S = np.zeros((20, 12), np.float32)
    for i in range(4):
        for l in range(5):
            S[l * 4 + i, i] = 1.0
        S[0 * 4 + i, 4 + i] = 1.0
        S[4 * 4 + i, 8 + i] = 1.0
    WSM = jnp.asarray(S) @ M
    cb = const.reshape(1, 128)
    Xn = jax.lax.reduce_precision(x.reshape(N, 20), 8, 7)

    rowE = jnp.concatenate([edge_index[0], jnp.zeros((EP - E,), jnp.int32)])
    colE = jnp.concatenate([edge_index[1], jnp.zeros((EP - E,), jnp.int32)])
    rowN = jnp.concatenate([edge_index[0, :N], jnp.zeros((NP2 - N,), jnp.int32)])

    r1 = lambda v: v.reshape(1, -1)

    p1, p2, p3 = _run_node_enc(Xn, node_type, WSM, cb, fuse_w[:128],
                               fuse_w[128:129], r1(fuse_b), e1_w1[:128],
                               e1_w1[128:256], n1_w1[:128])
    gr, gc, g3p = _sc_gather3(p1, p2, p3, rowE, colE, rowN)
    ea1 = _run_edge1(gr, gc, edge_attr, e1_w1[256:260], r1(e1_b1),
                     e1_w2, r1(e1_b2))
    q1, q2 = _run_node1(g3p, ea1, n1_w1[128:256], r1(n1_b1), n1_w2, r1(n1_b2),
                        e2_w1[:128], e2_w1[128:256])
    gr2, gc2 = _sc_gather2(q1, q2, rowE, colE)
    z, st = _run_edge2(gr2, gc2, ea1, e2_w1[256:384], r1(e2_b1), e2_w2,
                       r1(e2_b2), f_w1, r1(f_b1))
    mu = st[0] / E
    var = st[1] / E - mu * mu
    a = bn_g / jnp.sqrt(var + 1e-5)
    c = bn_b - mu * a
    return _run_final(z, r1(a), r1(c), f_w2, r1(f_b2))


# speedup vs baseline: 1.6804x; 1.6804x over previous
import functools

import jax
import jax.numpy as jnp
import numpy as np
from jax import lax
from jax.experimental import pallas as pl
from jax.experimental.pallas import tpu as pltpu
from jax.experimental.pallas import tpu_sc as plsc

N = 10000
E = 160000
BE = 1280
BN = 2000
CH = 128
NW = 32
NP = 10240

_f32 = jnp.float32
_HI = jax.lax.Precision.HIGHEST


def _dot(a, b):
    return jnp.dot(a, b, precision=_HI, preferred_element_type=_f32)


def _b1dot(a, b):
    return jnp.dot(a.astype(jnp.bfloat16), b.astype(jnp.bfloat16),
                   preferred_element_type=_f32)


def _bdot(a, b):
    return _b1dot(a, b)


def _node_enc_body(xn, nt, wsm, cb, fwa, fwb, fb, a1, b1w, n1a, p1, p2, p3):
    mc = _dot(xn[...], wsm[...]) + cb[...]
    h = _bdot(mc, fwa[...]) + _bdot(nt[...], fwb[...]) + fb[...]
    p1[...] = _bdot(h, a1[...])
    p2[...] = _bdot(h, b1w[...])
    p3[...] = _bdot(h, n1a[...])


def _edge1_body(gr, gc, ea, wc, b1, w2, b2, o):
    u = gr[...] + gc[...] + _bdot(ea[...], wc[...]) + b1[...]
    o[...] = _bdot(jax.nn.relu(u), w2[...]) + b2[...]


def _node1_body(g3, ea1n, n1b, n1b1, n1w2, n1b2, e2a, e2b, q1, q2):
    u = jax.nn.relu(g3[...] + _bdot(ea1n[...], n1b[...]) + n1b1[...])
    h1 = _bdot(u, n1w2[...]) + n1b2[...]
    q1[...] = _bdot(h1, e2a[...])
    q2[...] = _bdot(h1, e2b[...])


def _edge2_body(gr2, gc2, ea1, e2c, e2b1, e2w2, e2b2, fw1, fb1, z_ref, st_ref):
    u = jax.nn.relu(gr2[...] + gc2[...] +
                    _bdot(ea1[...], e2c[...]) + e2b1[...])
    v = _bdot(u, e2w2[...]) + e2b2[...]
    z = _bdot(v, fw1[...]) + fb1[...]
    z_ref[...] = z
    s = jnp.sum(z, axis=0).reshape(1, 128)
    sq = jnp.sum(z * z, axis=0).reshape(1, 128)
    upd = jnp.concatenate([s, sq, jnp.zeros((6, 128), _f32)], axis=0)

    @pl.when(pl.program_id(0) == 0)
    def _():
        st_ref[...] = jnp.zeros((8, 128), _f32)

    st_ref[...] += upd


def _final_body(z, av, cv, fw2, fb2, o):
    u = jax.nn.relu(z[...] * av[...] + cv[...])
    o[...] = _bdot(u, fw2[...]) + fb2[...]


def _w_spec(shape):
    return pl.BlockSpec(shape, lambda i: (0,) * len(shape))


def _run_node_enc(Xn, nt, WSM, cb, fwa, fwb, fb, a1, b1w, n1a):
    grid = (N // BN,)
    blk = lambda r, c: pl.BlockSpec((r, c), lambda i: (i, 0))
    return pl.pallas_call(
        _node_enc_body,
        grid=grid,
        in_specs=[blk(BN, 20), blk(BN, 1), _w_spec((20, 128)), _w_spec((1, 128)),
                  _w_spec((128, 128)), _w_spec((1, 128)), _w_spec((1, 128)),
                  _w_spec((128, 128)), _w_spec((128, 128)), _w_spec((128, 128))],
        out_specs=[blk(BN, 128)] * 3,
        out_shape=[jax.ShapeDtypeStruct((N, 128), _f32)] * 3,
    )(Xn, nt, WSM, cb, fwa, fwb, fb, a1, b1w, n1a)


def _run_edge1(gr, gc, ea, wc, b1, w2, b2):
    grid = (E // BE,)
    blk = lambda r, c: pl.BlockSpec((r, c), lambda i: (i, 0))
    return pl.pallas_call(
        _edge1_body,
        grid=grid,
        in_specs=[blk(BE, 128), blk(BE, 128), blk(BE, 4), _w_spec((4, 128)),
                  _w_spec((1, 128)), _w_spec((128, 128)), _w_spec((1, 128))],
        out_specs=blk(BE, 128),
        out_shape=jax.ShapeDtypeStruct((E, 128), _f32),
    )(gr, gc, ea, wc, b1, w2, b2)


def _run_node1(g3p, ea1, n1b, n1b1, n1w2, n1b2, e2a, e2b):
    grid = (N // BN,)
    blk = lambda r, c: pl.BlockSpec((r, c), lambda i: (i, 0))
    return pl.pallas_call(
        _node1_body,
        grid=grid,
        in_specs=[blk(BN, 128), blk(BN, 128), _w_spec((128, 128)), _w_spec((1, 128)),
                  _w_spec((128, 128)), _w_spec((1, 128)),
                  _w_spec((128, 128)), _w_spec((128, 128))],
        out_specs=[blk(BN, 128)] * 2,
        out_shape=[jax.ShapeDtypeStruct((N, 128), _f32)] * 2,
    )(g3p, ea1, n1b, n1b1, n1w2, n1b2, e2a, e2b)


def _run_edge2(gr2, gc2, ea1, e2c, e2b1, e2w2, e2b2, fw1, fb1):
    grid = (E // BE,)
    blk = lambda r, c: pl.BlockSpec((r, c), lambda i: (i, 0))
    return pl.pallas_call(
        _edge2_body,
        grid=grid,
        in_specs=[blk(BE, 128), blk(BE, 128), blk(BE, 128), _w_spec((128, 128)),
                  _w_spec((1, 128)), _w_spec((128, 128)), _w_spec((1, 128)),
                  _w_spec((128, 128)), _w_spec((1, 128))],
        out_specs=[blk(BE, 128), pl.BlockSpec((8, 128), lambda i: (0, 0))],
        out_shape=[jax.ShapeDtypeStruct((E, 128), _f32),
                   jax.ShapeDtypeStruct((8, 128), _f32)],
    )(gr2, gc2, ea1, e2c, e2b1, e2w2, e2b2, fw1, fb1)


def _run_final(z, av, cv, fw2, fb2):
    grid = (E // BE,)
    blk = lambda r, c: pl.BlockSpec((r, c), lambda i: (i, 0))
    return pl.pallas_call(
        _final_body,
        grid=grid,
        in_specs=[blk(BE, 128), _w_spec((1, 128)), _w_spec((1, 128)),
                  _w_spec((128, 3)), _w_spec((1, 3))],
        out_specs=blk(BE, 3),
        out_shape=jax.ShapeDtypeStruct((E, 3), _f32),
    )(z, av, cv, fw2, fb2)


def _sc_mesh():
    return plsc.VectorSubcoreMesh(core_axis_name="c", subcore_axis_name="s")


def _gather_chunks(wid, tbl, idx_hbm, out_hbm, nchunks, idx_v, rows_v, sem):
    iters = (nchunks + NW - 1) // NW

    @pl.loop(0, iters)
    def _(i):
        chunk = i * NW + wid

        @pl.when(chunk < nchunks)
        def _():
            base = chunk * CH
            pltpu.sync_copy(idx_hbm.at[pl.ds(base, CH)], idx_v)
            pltpu.async_copy(tbl.at[idx_v], rows_v, sem).wait()
            pltpu.sync_copy(rows_v, out_hbm.at[pl.ds(base, CH)])


_SC_SCRATCH = [pltpu.VMEM((CH,), jnp.int32),
               pltpu.VMEM((CH, 128), _f32),
               pltpu.SemaphoreType.DMA]


def _sc_gather3(p1, p2, p3, rowE, colE, rowN):
    @functools.partial(
        pl.kernel,
        mesh=_sc_mesh(),
        out_type=(jax.ShapeDtypeStruct((E, 128), _f32),
                  jax.ShapeDtypeStruct((E, 128), _f32),
                  jax.ShapeDtypeStruct((NP, 128), _f32)),
        scratch_types=list(_SC_SCRATCH),
    )
    def k(p1_h, p2_h, p3_h, row_h, col_h, rown_h, gr_h, gc_h, g3_h,
          idx_v, rows_v, sem):
        wid = lax.axis_index("s") * 2 + lax.axis_index("c")
        _gather_chunks(wid, p1_h, row_h, gr_h, E // CH, idx_v, rows_v, sem)
        _gather_chunks(wid, p2_h, col_h, gc_h, E // CH, idx_v, rows_v, sem)
        _gather_chunks(wid, p3_h, rown_h, g3_h, NP // CH, idx_v, rows_v, sem)

    return k(p1, p2, p3, rowE, colE, rowN)


def _sc_gather2(q1, q2, rowE, colE):
    @functools.partial(
        pl.kernel,
        mesh=_sc_mesh(),
        out_type=(jax.ShapeDtypeStruct((E, 128), _f32),
                  jax.ShapeDtypeStruct((E, 128), _f32)),
        scratch_types=list(_SC_SCRATCH),
    )
    def k(q1_h, q2_h, row_h, col_h, gr_h, gc_h, idx_v, rows_v, sem):
        wid = lax.axis_index("s") * 2 + lax.axis_index("c")
        _gather_chunks(wid, q1_h, row_h, gr_h, E // CH, idx_v, rows_v, sem)
        _gather_chunks(wid, q2_h, col_h, gc_h, E // CH, idx_v, rows_v, sem)

    return k(q1, q2, rowE, colE)


def kernel(x, edge_index, edge_attr, batch, node_type, emb, conv_w, conv_b,
           fuse_w, fuse_b, e1_w1, e1_b1, e1_w2, e1_b2, n1_w1, n1_b1, n1_w2,
           n1_b2, e2_w1, e2_b1, e2_w2, e2_b2, n2_w1, n2_b1, n2_w2, n2_b2,
           f_w1, f_b1, bn_g, bn_b, f_w2, f_b2):
    L = 5
    _b = lambda v: jax.lax.reduce_precision(v, 8, 7)
    cwb = _b(conv_w)
    embb = _b(emb)
    w0 = cwb[:, :, 0]; w1 = cwb[:, :, 1]; w2 = cwb[:, :, 2]
    ws = w0 + w1 + w2
    M = jnp.concatenate([ws[:, :4].T, -w2[:, :4].T, -w0[:, :4].T], axis=0) / L
    s_pe = embb.sum(0)
    const = (s_pe @ ws[:, 4:].T - embb[4] @ w0[:, 4:].T - embb[0] @ w2[:, 4:].T) / L + conv_b
    S = np.zeros((20, 12), np.float32)
    for i in range(4):
        for l in range(5):
            S[l * 4 + i, i] = 1.0
        S[0 * 4 + i, 4 + i] = 1.0
        S[4 * 4 + i, 8 + i] = 1.0
    WSM = jnp.asarray(S) @ M
    cb = const.reshape(1, 128)
    Xn = jax.lax.reduce_precision(x.reshape(N, 20), 8, 7)

    rowE = edge_index[0]
    colE = edge_index[1]
    rowN = jnp.concatenate([edge_index[0, :N], jnp.zeros((NP - N,), jnp.int32)])

    r1 = lambda v: v.reshape(1, -1)

    p1, p2, p3 = _run_node_enc(Xn, node_type, WSM, cb, fuse_w[:128],
                               fuse_w[128:129], r1(fuse_b), e1_w1[:128],
                               e1_w1[128:256], n1_w1[:128])
    gr, gc, g3p = _sc_gather3(p1, p2, p3, rowE, colE, rowN)
    ea1 = _run_edge1(gr, gc, edge_attr, e1_w1[256:260], r1(e1_b1),
                     e1_w2, r1(e1_b2))
    q1, q2 = _run_node1(g3p, ea1, n1_w1[128:256], r1(n1_b1), n1_w2, r1(n1_b2),
                        e2_w1[:128], e2_w1[128:256])
    gr2, gc2 = _sc_gather2(q1, q2, rowE, colE)
    z, st = _run_edge2(gr2, gc2, ea1, e2_w1[256:384], r1(e2_b1), e2_w2,
                       r1(e2_b2), f_w1, r1(f_b1))
    mu = st[0] / E
    var = st[1] / E - mu * mu
    a = bn_g / jnp.sqrt(var + 1e-5)
    c = bn_b - mu * a
    return _run_final(z, r1(a), r1(c), f_w2, r1(f_b2))


# speedup vs baseline: 1.8611x; 1.1075x over previous
import functools

import jax
import jax.numpy as jnp
import numpy as np
from jax import lax
from jax.experimental import pallas as pl
from jax.experimental.pallas import tpu as pltpu
from jax.experimental.pallas import tpu_sc as plsc

N = 10000
E = 160000
BE = 1280
BN = 2000
CH = 128
NW = 32
NP = 10240

_f32 = jnp.float32
_HI = jax.lax.Precision.HIGHEST


def _dot(a, b):
    return jnp.dot(a, b, precision=_HI, preferred_element_type=_f32)


def _b1dot(a, b):
    return jnp.dot(a.astype(jnp.bfloat16), b.astype(jnp.bfloat16),
                   preferred_element_type=_f32)


def _bdot(a, b):
    return _b1dot(a, b)


def _node_enc_body(xn, nt, wsm, cb, fwa, fwb, fb, a1, b1w, n1a, p1, p2, p3):
    mc = _dot(xn[...], wsm[...]) + cb[...]
    h = _bdot(mc, fwa[...]) + _bdot(nt[...], fwb[...]) + fb[...]
    p1[...] = _bdot(h, a1[...])
    p2[...] = _bdot(h, b1w[...])
    p3[...] = _bdot(h, n1a[...])


def _edge1_body(gr, gc, ea, wc, b1, w2, b2, o):
    u = gr[...] + gc[...] + _bdot(ea[...], wc[...]) + b1[...]
    o[...] = _bdot(jax.nn.relu(u), w2[...]) + b2[...]


def _node1_body(g3, ea1n, n1b, n1b1, n1w2, n1b2, e2a, e2b, q1, q2):
    u = jax.nn.relu(g3[...] + _bdot(ea1n[...], n1b[...]) + n1b1[...])
    h1 = _bdot(u, n1w2[...]) + n1b2[...]
    q1[...] = _bdot(h1, e2a[...])
    q2[...] = _bdot(h1, e2b[...])


def _edge2_body(gr2, gc2, ea1, e2c, e2b1, e2w2, e2b2, fw1, fb1, z_ref, st_ref):
    u = jax.nn.relu(gr2[...] + gc2[...] +
                    _bdot(ea1[...], e2c[...]) + e2b1[...])
    v = _bdot(u, e2w2[...]) + e2b2[...]
    z = _bdot(v, fw1[...]) + fb1[...]
    z_ref[...] = z
    s = jnp.sum(z, axis=0).reshape(1, 128)
    sq = jnp.sum(z * z, axis=0).reshape(1, 128)
    upd = jnp.concatenate([s, sq, jnp.zeros((6, 128), _f32)], axis=0)

    @pl.when(pl.program_id(0) == 0)
    def _():
        st_ref[...] = jnp.zeros((8, 128), _f32)

    st_ref[...] += upd


def _final_body(z, av, cv, fw2, fb2, o):
    u = jax.nn.relu(z[...] * av[...] + cv[...])
    o[...] = _bdot(u, fw2[...]) + fb2[...]


def _w_spec(shape):
    return pl.BlockSpec(shape, lambda i: (0,) * len(shape))


def _run_node_enc(Xn, nt, WSM, cb, fwa, fwb, fb, a1, b1w, n1a):
    grid = (N // BN,)
    blk = lambda r, c: pl.BlockSpec((r, c), lambda i: (i, 0))
    return pl.pallas_call(
        _node_enc_body,
        grid=grid,
        in_specs=[blk(BN, 20), blk(BN, 1), _w_spec((20, 128)), _w_spec((1, 128)),
                  _w_spec((128, 128)), _w_spec((1, 128)), _w_spec((1, 128)),
                  _w_spec((128, 128)), _w_spec((128, 128)), _w_spec((128, 128))],
        out_specs=[blk(BN, 128)] * 3,
        out_shape=[jax.ShapeDtypeStruct((N, 128), _f32)] * 3,
    )(Xn, nt, WSM, cb, fwa, fwb, fb, a1, b1w, n1a)


def _run_edge1(gr, gc, ea, wc, b1, w2, b2):
    grid = (E // BE,)
    blk = lambda r, c: pl.BlockSpec((r, c), lambda i: (i, 0))
    return pl.pallas_call(
        _edge1_body,
        grid=grid,
        in_specs=[blk(BE, 128), blk(BE, 128), blk(BE, 4), _w_spec((4, 128)),
                  _w_spec((1, 128)), _w_spec((128, 128)), _w_spec((1, 128))],
        out_specs=blk(BE, 128),
        out_shape=jax.ShapeDtypeStruct((E, 128), _f32),
    )(gr, gc, ea, wc, b1, w2, b2)


def _run_node1(g3p, ea1, n1b, n1b1, n1w2, n1b2, e2a, e2b):
    grid = (N // BN,)
    blk = lambda r, c: pl.BlockSpec((r, c), lambda i: (i, 0))
    return pl.pallas_call(
        _node1_body,
        grid=grid,
        in_specs=[blk(BN, 128), blk(BN, 128), _w_spec((128, 128)), _w_spec((1, 128)),
                  _w_spec((128, 128)), _w_spec((1, 128)),
                  _w_spec((128, 128)), _w_spec((128, 128))],
        out_specs=[blk(BN, 128)] * 2,
        out_shape=[jax.ShapeDtypeStruct((N, 128), _f32)] * 2,
    )(g3p, ea1, n1b, n1b1, n1w2, n1b2, e2a, e2b)


def _run_edge2(gr2, gc2, ea1, e2c, e2b1, e2w2, e2b2, fw1, fb1):
    grid = (E // BE,)
    blk = lambda r, c: pl.BlockSpec((r, c), lambda i: (i, 0))
    return pl.pallas_call(
        _edge2_body,
        grid=grid,
        in_specs=[blk(BE, 128), blk(BE, 128), blk(BE, 128), _w_spec((128, 128)),
                  _w_spec((1, 128)), _w_spec((128, 128)), _w_spec((1, 128)),
                  _w_spec((128, 128)), _w_spec((1, 128))],
        out_specs=[blk(BE, 128), pl.BlockSpec((8, 128), lambda i: (0, 0))],
        out_shape=[jax.ShapeDtypeStruct((E, 128), _f32),
                   jax.ShapeDtypeStruct((8, 128), _f32)],
    )(gr2, gc2, ea1, e2c, e2b1, e2w2, e2b2, fw1, fb1)


def _run_final(z, av, cv, fw2, fb2):
    grid = (E // BE,)
    blk = lambda r, c: pl.BlockSpec((r, c), lambda i: (i, 0))
    return pl.pallas_call(
        _final_body,
        grid=grid,
        in_specs=[blk(BE, 128), _w_spec((1, 128)), _w_spec((1, 128)),
                  _w_spec((128, 3)), _w_spec((1, 3))],
        out_specs=blk(BE, 3),
        out_shape=jax.ShapeDtypeStruct((E, 3), _f32),
    )(z, av, cv, fw2, fb2)


def _sc_mesh():
    return plsc.VectorSubcoreMesh(core_axis_name="c", subcore_axis_name="s")


def _gather_chunks(wid, tbl, idx_hbm, out_hbm, nchunks, idx_v, rows_v, sem):
    npairs = nchunks // 2
    iters = (npairs + NW - 1) // NW

    @pl.loop(0, iters)
    def _(i):
        pair = i * NW + wid

        @pl.when(pair < npairs)
        def _():
            base = pair * (2 * CH)
            pltpu.sync_copy(idx_hbm.at[pl.ds(base, 2 * CH)], idx_v)
            h1 = pltpu.async_copy(tbl.at[idx_v.at[pl.ds(0, CH)]],
                                  rows_v.at[pl.ds(0, CH)], sem)
            h2 = pltpu.async_copy(tbl.at[idx_v.at[pl.ds(CH, CH)]],
                                  rows_v.at[pl.ds(CH, CH)], sem)
            h1.wait()
            h2.wait()
            pltpu.sync_copy(rows_v, out_hbm.at[pl.ds(base, 2 * CH)])


_SC_SCRATCH = [pltpu.VMEM((2 * CH,), jnp.int32),
               pltpu.VMEM((2 * CH, 128), _f32),
               pltpu.SemaphoreType.DMA]


def _sc_gather3(p1, p2, p3, rowE, colE, rowN):
    @functools.partial(
        pl.kernel,
        mesh=_sc_mesh(),
        out_type=(jax.ShapeDtypeStruct((E, 128), _f32),
                  jax.ShapeDtypeStruct((E, 128), _f32),
                  jax.ShapeDtypeStruct((NP, 128), _f32)),
        scratch_types=list(_SC_SCRATCH),
    )
    def k(p1_h, p2_h, p3_h, row_h, col_h, rown_h, gr_h, gc_h, g3_h,
          idx_v, rows_v, sem):
        wid = lax.axis_index("s") * 2 + lax.axis_index("c")
        _gather_chunks(wid, p1_h, row_h, gr_h, E // CH, idx_v, rows_v, sem)
        _gather_chunks(wid, p2_h, col_h, gc_h, E // CH, idx_v, rows_v, sem)
        _gather_chunks(wid, p3_h, rown_h, g3_h, NP // CH, idx_v, rows_v, sem)

    return k(p1, p2, p3, rowE, colE, rowN)


def _sc_gather2(q1, q2, rowE, colE):
    @functools.partial(
        pl.kernel,
        mesh=_sc_mesh(),
        out_type=(jax.ShapeDtypeStruct((E, 128), _f32),
                  jax.ShapeDtypeStruct((E, 128), _f32)),
        scratch_types=list(_SC_SCRATCH),
    )
    def k(q1_h, q2_h, row_h, col_h, gr_h, gc_h, idx_v, rows_v, sem):
        wid = lax.axis_index("s") * 2 + lax.axis_index("c")
        _gather_chunks(wid, q1_h, row_h, gr_h, E // CH, idx_v, rows_v, sem)
        _gather_chunks(wid, q2_h, col_h, gc_h, E // CH, idx_v, rows_v, sem)

    return k(q1, q2, rowE, colE)


def kernel(x, edge_index, edge_attr, batch, node_type, emb, conv_w, conv_b,
           fuse_w, fuse_b, e1_w1, e1_b1, e1_w2, e1_b2, n1_w1, n1_b1, n1_w2,
           n1_b2, e2_w1, e2_b1, e2_w2, e2_b2, n2_w1, n2_b1, n2_w2, n2_b2,
           f_w1, f_b1, bn_g, bn_b, f_w2, f_b2):
    L = 5
    _b = lambda v: jax.lax.reduce_precision(v, 8, 7)
    cwb = _b(conv_w)
    embb = _b(emb)
    w0 = cwb[:, :, 0]; w1 = cwb[:, :, 1]; w2 = cwb[:, :, 2]
    ws = w0 + w1 + w2
    M = jnp.concatenate([ws[:, :4].T, -w2[:, :4].T, -w0[:, :4].T], axis=0) / L
    s_pe = embb.sum(0)
    const = (s_pe @ ws[:, 4:].T - embb[4] @ w0[:, 4:].T - embb[0] @ w2[:, 4:].T) / L + conv_b
    S = np.zeros((20, 12), np.float32)
    for i in range(4):
        for l in range(5):
            S[l * 4 + i, i] = 1.0
        S[0 * 4 + i, 4 + i] = 1.0
        S[4 * 4 + i, 8 + i] = 1.0
    WSM = jnp.asarray(S) @ M
    cb = const.reshape(1, 128)
    Xn = jax.lax.reduce_precision(x.reshape(N, 20), 8, 7)

    rowE = edge_index[0]
    colE = edge_index[1]
    rowN = jnp.concatenate([edge_index[0, :N], jnp.zeros((NP - N,), jnp.int32)])

    r1 = lambda v: v.reshape(1, -1)

    p1, p2, p3 = _run_node_enc(Xn, node_type, WSM, cb, fuse_w[:128],
                               fuse_w[128:129], r1(fuse_b), e1_w1[:128],
                               e1_w1[128:256], n1_w1[:128])
    gr, gc, g3p = _sc_gather3(p1, p2, p3, rowE, colE, rowN)
    ea1 = _run_edge1(gr, gc, edge_attr, e1_w1[256:260], r1(e1_b1),
                     e1_w2, r1(e1_b2))
    q1, q2 = _run_node1(g3p, ea1, n1_w1[128:256], r1(n1_b1), n1_w2, r1(n1_b2),
                        e2_w1[:128], e2_w1[128:256])
    gr2, gc2 = _sc_gather2(q1, q2, rowE, colE)
    z, st = _run_edge2(gr2, gc2, ea1, e2_w1[256:384], r1(e2_b1), e2_w2,
                       r1(e2_b2), f_w1, r1(f_b1))
    mu = st[0] / E
    var = st[1] / E - mu * mu
    a = bn_g / jnp.sqrt(var + 1e-5)
    c = bn_b - mu * a
    return _run_final(z, r1(a), r1(c), f_w2, r1(f_b2))


# speedup vs baseline: 1.9371x; 1.0409x over previous
import functools

import jax
import jax.numpy as jnp
import numpy as np
from jax import lax
from jax.experimental import pallas as pl
from jax.experimental.pallas import tpu as pltpu
from jax.experimental.pallas import tpu_sc as plsc

N = 10000
E = 160000
BE = 1280
BN = 2000
CH = 128
NW = 32
NP = 10240

_f32 = jnp.float32
_HI = jax.lax.Precision.HIGHEST


def _dot(a, b):
    return jnp.dot(a, b, precision=_HI, preferred_element_type=_f32)


def _b1dot(a, b):
    return jnp.dot(a.astype(jnp.bfloat16), b.astype(jnp.bfloat16),
                   preferred_element_type=_f32)


def _bdot(a, b):
    return _b1dot(a, b)


def _node_enc_body(xn, nt, wsm, cb, fwa, fwb, fb, a1, b1w, n1a, p1, p2, p3):
    mc = _dot(xn[...], wsm[...]) + cb[...]
    h = _bdot(mc, fwa[...]) + _bdot(nt[...], fwb[...]) + fb[...]
    p1[...] = _bdot(h, a1[...])
    p2[...] = _bdot(h, b1w[...])
    p3[...] = _bdot(h, n1a[...])


def _edge1_body(gr, gc, ea, wc, b1, w2, b2, o):
    u = gr[...] + gc[...] + _bdot(ea[...], wc[...]) + b1[...]
    o[...] = _bdot(jax.nn.relu(u), w2[...]) + b2[...]


def _node1_body(g3, ea1n, n1b, n1b1, n1w2, n1b2, e2a, e2b, q1, q2):
    u = jax.nn.relu(g3[...] + _bdot(ea1n[...], n1b[...]) + n1b1[...])
    h1 = _bdot(u, n1w2[...]) + n1b2[...]
    q1[...] = _bdot(h1, e2a[...])
    q2[...] = _bdot(h1, e2b[...])


def _edge2_body(gr2, gc2, ea1, e2c, e2b1, e2w2, e2b2, fw1, fb1, z_ref, st_ref):
    u = jax.nn.relu(gr2[...] + gc2[...] +
                    _bdot(ea1[...], e2c[...]) + e2b1[...])
    v = _bdot(u, e2w2[...]) + e2b2[...]
    z = _bdot(v, fw1[...]) + fb1[...]
    z_ref[...] = z
    s = jnp.sum(z, axis=0).reshape(1, 128)
    sq = jnp.sum(z * z, axis=0).reshape(1, 128)
    upd = jnp.concatenate([s, sq, jnp.zeros((6, 128), _f32)], axis=0)

    @pl.when(pl.program_id(0) == 0)
    def _():
        st_ref[...] = jnp.zeros((8, 128), _f32)

    st_ref[...] += upd


def _final_body(z, av, cv, fw2, fb2, o):
    u = jax.nn.relu(z[...] * av[...] + cv[...])
    o[...] = _bdot(u, fw2[...]) + fb2[...]


def _w_spec(shape):
    return pl.BlockSpec(shape, lambda i: (0,) * len(shape))


def _run_node_enc(Xn, nt, WSM, cb, fwa, fwb, fb, a1, b1w, n1a):
    grid = (N // BN,)
    blk = lambda r, c: pl.BlockSpec((r, c), lambda i: (i, 0))
    return pl.pallas_call(
        _node_enc_body,
        grid=grid,
        in_specs=[blk(BN, 20), blk(BN, 1), _w_spec((20, 128)), _w_spec((1, 128)),
                  _w_spec((128, 128)), _w_spec((1, 128)), _w_spec((1, 128)),
                  _w_spec((128, 128)), _w_spec((128, 128)), _w_spec((128, 128))],
        out_specs=[blk(BN, 128)] * 3,
        out_shape=[jax.ShapeDtypeStruct((N, 128), _f32)] * 3,
    )(Xn, nt, WSM, cb, fwa, fwb, fb, a1, b1w, n1a)


def _run_edge1(gr, gc, ea, wc, b1, w2, b2):
    grid = (E // BE,)
    blk = lambda r, c: pl.BlockSpec((r, c), lambda i: (i, 0))
    return pl.pallas_call(
        _edge1_body,
        grid=grid,
        in_specs=[blk(BE, 128), blk(BE, 128), blk(BE, 4), _w_spec((4, 128)),
                  _w_spec((1, 128)), _w_spec((128, 128)), _w_spec((1, 128))],
        out_specs=blk(BE, 128),
        out_shape=jax.ShapeDtypeStruct((E, 128), _f32),
    )(gr, gc, ea, wc, b1, w2, b2)


def _run_node1(g3p, ea1, n1b, n1b1, n1w2, n1b2, e2a, e2b):
    grid = (N // BN,)
    blk = lambda r, c: pl.BlockSpec((r, c), lambda i: (i, 0))
    return pl.pallas_call(
        _node1_body,
        grid=grid,
        in_specs=[blk(BN, 128), blk(BN, 128), _w_spec((128, 128)), _w_spec((1, 128)),
                  _w_spec((128, 128)), _w_spec((1, 128)),
                  _w_spec((128, 128)), _w_spec((128, 128))],
        out_specs=[blk(BN, 128)] * 2,
        out_shape=[jax.ShapeDtypeStruct((N, 128), _f32)] * 2,
    )(g3p, ea1, n1b, n1b1, n1w2, n1b2, e2a, e2b)


def _run_edge2(gr2, gc2, ea1, e2c, e2b1, e2w2, e2b2, fw1, fb1):
    grid = (E // BE,)
    blk = lambda r, c: pl.BlockSpec((r, c), lambda i: (i, 0))
    return pl.pallas_call(
        _edge2_body,
        grid=grid,
        in_specs=[blk(BE, 128), blk(BE, 128), blk(BE, 128), _w_spec((128, 128)),
                  _w_spec((1, 128)), _w_spec((128, 128)), _w_spec((1, 128)),
                  _w_spec((128, 128)), _w_spec((1, 128))],
        out_specs=[blk(BE, 128), pl.BlockSpec((8, 128), lambda i: (0, 0))],
        out_shape=[jax.ShapeDtypeStruct((E, 128), _f32),
                   jax.ShapeDtypeStruct((8, 128), _f32)],
    )(gr2, gc2, ea1, e2c, e2b1, e2w2, e2b2, fw1, fb1)


def _run_final(z, av, cv, fw2, fb2):
    grid = (E // BE,)
    blk = lambda r, c: pl.BlockSpec((r, c), lambda i: (i, 0))
    return pl.pallas_call(
        _final_body,
        grid=grid,
        in_specs=[blk(BE, 128), _w_spec((1, 128)), _w_spec((1, 128)),
                  _w_spec((128, 3)), _w_spec((1, 3))],
        out_specs=blk(BE, 3),
        out_shape=jax.ShapeDtypeStruct((E, 3), _f32),
    )(z, av, cv, fw2, fb2)


def _sc_mesh():
    return plsc.VectorSubcoreMesh(core_axis_name="c", subcore_axis_name="s")


def _gather_pair(tbl, idx_hbm, out_hbm, base, idx_v, rows_v, sem, k):
    pltpu.sync_copy(idx_hbm.at[pl.ds(base, k * CH)], idx_v.at[pl.ds(0, k * CH)])
    hs = [pltpu.async_copy(tbl.at[idx_v.at[pl.ds(j * CH, CH)]],
                           rows_v.at[pl.ds(j * CH, CH)], sem)
          for j in range(k)]
    for h in hs:
        h.wait()
    pltpu.sync_copy(rows_v.at[pl.ds(0, k * CH)], out_hbm.at[pl.ds(base, k * CH)])


def _gather_chunks(wid, tbl, idx_hbm, out_hbm, nchunks, idx_v, rows_v, sem):
    nquads = nchunks // 4
    iters = (nquads + NW - 1) // NW

    @pl.loop(0, iters)
    def _(i):
        quad = i * NW + wid

        @pl.when(quad < nquads)
        def _():
            _gather_pair(tbl, idx_hbm, out_hbm, quad * (4 * CH),
                         idx_v, rows_v, sem, 4)

    rem = nchunks - nquads * 4
    if rem:
        @pl.when(wid == 0)
        def _():
            _gather_pair(tbl, idx_hbm, out_hbm, nquads * (4 * CH),
                         idx_v, rows_v, sem, rem)


_SC_SCRATCH = [pltpu.VMEM((4 * CH,), jnp.int32),
               pltpu.VMEM((4 * CH, 128), _f32),
               pltpu.SemaphoreType.DMA]


def _sc_gather3(p1, p2, p3, rowE, colE, rowN):
    @functools.partial(
        pl.kernel,
        mesh=_sc_mesh(),
        out_type=(jax.ShapeDtypeStruct((E, 128), _f32),
                  jax.ShapeDtypeStruct((E, 128), _f32),
                  jax.ShapeDtypeStruct((NP, 128), _f32)),
        scratch_types=list(_SC_SCRATCH),
    )
    def k(p1_h, p2_h, p3_h, row_h, col_h, rown_h, gr_h, gc_h, g3_h,
          idx_v, rows_v, sem):
        wid = lax.axis_index("s") * 2 + lax.axis_index("c")
        _gather_chunks(wid, p1_h, row_h, gr_h, E // CH, idx_v, rows_v, sem)
        _gather_chunks(wid, p2_h, col_h, gc_h, E // CH, idx_v, rows_v, sem)
        _gather_chunks(wid, p3_h, rown_h, g3_h, NP // CH, idx_v, rows_v, sem)

    return k(p1, p2, p3, rowE, colE, rowN)


def _sc_gather2(q1, q2, rowE, colE):
    @functools.partial(
        pl.kernel,
        mesh=_sc_mesh(),
        out_type=(jax.ShapeDtypeStruct((E, 128), _f32),
                  jax.ShapeDtypeStruct((E, 128), _f32)),
        scratch_types=list(_SC_SCRATCH),
    )
    def k(q1_h, q2_h, row_h, col_h, gr_h, gc_h, idx_v, rows_v, sem):
        wid = lax.axis_index("s") * 2 + lax.axis_index("c")
        _gather_chunks(wid, q1_h, row_h, gr_h, E // CH, idx_v, rows_v, sem)
        _gather_chunks(wid, q2_h, col_h, gc_h, E // CH, idx_v, rows_v, sem)

    return k(q1, q2, rowE, colE)


def kernel(x, edge_index, edge_attr, batch, node_type, emb, conv_w, conv_b,
           fuse_w, fuse_b, e1_w1, e1_b1, e1_w2, e1_b2, n1_w1, n1_b1, n1_w2,
           n1_b2, e2_w1, e2_b1, e2_w2, e2_b2, n2_w1, n2_b1, n2_w2, n2_b2,
           f_w1, f_b1, bn_g, bn_b, f_w2, f_b2):
    L = 5
    _b = lambda v: jax.lax.reduce_precision(v, 8, 7)
    cwb = _b(conv_w)
    embb = _b(emb)
    w0 = cwb[:, :, 0]; w1 = cwb[:, :, 1]; w2 = cwb[:, :, 2]
    ws = w0 + w1 + w2
    M = jnp.concatenate([ws[:, :4].T, -w2[:, :4].T, -w0[:, :4].T], axis=0) / L
    s_pe = embb.sum(0)
    const = (s_pe @ ws[:, 4:].T - embb[4] @ w0[:, 4:].T - embb[0] @ w2[:, 4:].T) / L + conv_b
    S = np.zeros((20, 12), np.float32)
    for i in range(4):
        for l in range(5):
            S[l * 4 + i, i] = 1.0
        S[0 * 4 + i, 4 + i] = 1.0
        S[4 * 4 + i, 8 + i] = 1.0
    WSM = jnp.asarray(S) @ M
    cb = const.reshape(1, 128)
    Xn = jax.lax.reduce_precision(x.reshape(N, 20), 8, 7)

    rowE = edge_index[0]
    colE = edge_index[1]
    rowN = jnp.concatenate([edge_index[0, :N], jnp.zeros((NP - N,), jnp.int32)])

    r1 = lambda v: v.reshape(1, -1)

    p1, p2, p3 = _run_node_enc(Xn, node_type, WSM, cb, fuse_w[:128],
                               fuse_w[128:129], r1(fuse_b), e1_w1[:128],
                               e1_w1[128:256], n1_w1[:128])
    gr, gc, g3p = _sc_gather3(p1, p2, p3, rowE, colE, rowN)
    ea1 = _run_edge1(gr, gc, edge_attr, e1_w1[256:260], r1(e1_b1),
                     e1_w2, r1(e1_b2))
    q1, q2 = _run_node1(g3p, ea1, n1_w1[128:256], r1(n1_b1), n1_w2, r1(n1_b2),
                        e2_w1[:128], e2_w1[128:256])
    gr2, gc2 = _sc_gather2(q1, q2, rowE, colE)
    z, st = _run_edge2(gr2, gc2, ea1, e2_w1[256:384], r1(e2_b1), e2_w2,
                       r1(e2_b2), f_w1, r1(f_b1))
    mu = st[0] / E
    var = st[1] / E - mu * mu
    a = bn_g / jnp.sqrt(var + 1e-5)
    c = bn_b - mu * a
    return _run_final(z, r1(a), r1(c), f_w2, r1(f_b2))


# speedup vs baseline: 1.9398x; 1.0014x over previous
import functools

import jax
import jax.numpy as jnp
import numpy as np
from jax import lax
from jax.experimental import pallas as pl
from jax.experimental.pallas import tpu as pltpu
from jax.experimental.pallas import tpu_sc as plsc

N = 10000
E = 160000
BE = 1280
BN = 2000
CH = 128
NW = 32
NP = 10240

_f32 = jnp.float32
_HI = jax.lax.Precision.HIGHEST


def _dot(a, b):
    return jnp.dot(a, b, precision=_HI, preferred_element_type=_f32)


def _b1dot(a, b):
    return jnp.dot(a.astype(jnp.bfloat16), b.astype(jnp.bfloat16),
                   preferred_element_type=_f32)


def _bdot(a, b):
    return _b1dot(a, b)


def _node_enc_body(xn, nt, wsm, cb, fwa, fwb, fb, a1, b1w, n1a, p1, p2, p3):
    mc = _dot(xn[...], wsm[...]) + cb[...]
    h = _bdot(mc, fwa[...]) + _bdot(nt[...], fwb[...]) + fb[...]
    p1[...] = _bdot(h, a1[...])
    p2[...] = _bdot(h, b1w[...])
    p3[...] = _bdot(h, n1a[...])


def _edge1_body(gr, gc, ea, wc, b1, w2, b2, o):
    u = gr[...] + gc[...] + _bdot(ea[...], wc[...]) + b1[...]
    o[...] = _bdot(jax.nn.relu(u), w2[...]) + b2[...]


def _node1_body(g3, ea1n, n1b, n1b1, n1w2, n1b2, e2a, e2b, q1, q2):
    u = jax.nn.relu(g3[...] + _bdot(ea1n[...], n1b[...]) + n1b1[...])
    h1 = _bdot(u, n1w2[...]) + n1b2[...]
    q1[...] = _bdot(h1, e2a[...])
    q2[...] = _bdot(h1, e2b[...])


def _edge2_body(gr2, gc2, ea1, e2c, e2b1, e2w2, e2b2, fw1, fb1, z_ref, st_ref):
    u = jax.nn.relu(gr2[...] + gc2[...] +
                    _bdot(ea1[...], e2c[...]) + e2b1[...])
    v = _bdot(u, e2w2[...]) + e2b2[...]
    z = _bdot(v, fw1[...]) + fb1[...]
    z_ref[...] = z
    s = jnp.sum(z, axis=0).reshape(1, 128)
    sq = jnp.sum(z * z, axis=0).reshape(1, 128)
    st_ref[...] = jnp.concatenate([s, sq, jnp.zeros((6, 128), _f32)], axis=0)


def _final_body(z, av, cv, fw2, fb2, o):
    u = jax.nn.relu(z[...] * av[...] + cv[...])
    o[...] = _bdot(u, fw2[...]) + fb2[...]


def _w_spec(shape):
    return pl.BlockSpec(shape, lambda i: (0,) * len(shape))


def _run_node_enc(Xn, nt, WSM, cb, fwa, fwb, fb, a1, b1w, n1a):
    grid = (N // BN,)
    blk = lambda r, c: pl.BlockSpec((r, c), lambda i: (i, 0))
    return pl.pallas_call(
        _node_enc_body,
        grid=grid,
        in_specs=[blk(BN, 20), blk(BN, 1), _w_spec((20, 128)), _w_spec((1, 128)),
                  _w_spec((128, 128)), _w_spec((1, 128)), _w_spec((1, 128)),
                  _w_spec((128, 128)), _w_spec((128, 128)), _w_spec((128, 128))],
        out_specs=[blk(BN, 128)] * 3,
        out_shape=[jax.ShapeDtypeStruct((N, 128), _f32)] * 3,
        compiler_params=pltpu.CompilerParams(
            dimension_semantics=("parallel",)),
    )(Xn, nt, WSM, cb, fwa, fwb, fb, a1, b1w, n1a)


def _run_edge1(gr, gc, ea, wc, b1, w2, b2):
    grid = (E // BE,)
    blk = lambda r, c: pl.BlockSpec((r, c), lambda i: (i, 0))
    return pl.pallas_call(
        _edge1_body,
        grid=grid,
        in_specs=[blk(BE, 128), blk(BE, 128), blk(BE, 4), _w_spec((4, 128)),
                  _w_spec((1, 128)), _w_spec((128, 128)), _w_spec((1, 128))],
        out_specs=blk(BE, 128),
        out_shape=jax.ShapeDtypeStruct((E, 128), _f32),
        compiler_params=pltpu.CompilerParams(
            dimension_semantics=("parallel",)),
    )(gr, gc, ea, wc, b1, w2, b2)


def _run_node1(g3p, ea1, n1b, n1b1, n1w2, n1b2, e2a, e2b):
    grid = (N // BN,)
    blk = lambda r, c: pl.BlockSpec((r, c), lambda i: (i, 0))
    return pl.pallas_call(
        _node1_body,
        grid=grid,
        in_specs=[blk(BN, 128), blk(BN, 128), _w_spec((128, 128)), _w_spec((1, 128)),
                  _w_spec((128, 128)), _w_spec((1, 128)),
                  _w_spec((128, 128)), _w_spec((128, 128))],
        out_specs=[blk(BN, 128)] * 2,
        out_shape=[jax.ShapeDtypeStruct((N, 128), _f32)] * 2,
        compiler_params=pltpu.CompilerParams(
            dimension_semantics=("parallel",)),
    )(g3p, ea1, n1b, n1b1, n1w2, n1b2, e2a, e2b)


def _run_edge2(gr2, gc2, ea1, e2c, e2b1, e2w2, e2b2, fw1, fb1):
    grid = (E // BE,)
    blk = lambda r, c: pl.BlockSpec((r, c), lambda i: (i, 0))
    return pl.pallas_call(
        _edge2_body,
        grid=grid,
        in_specs=[blk(BE, 128), blk(BE, 128), blk(BE, 128), _w_spec((128, 128)),
                  _w_spec((1, 128)), _w_spec((128, 128)), _w_spec((1, 128)),
                  _w_spec((128, 128)), _w_spec((1, 128))],
        out_specs=[blk(BE, 128), pl.BlockSpec((8, 128), lambda i: (i, 0))],
        out_shape=[jax.ShapeDtypeStruct((E, 128), _f32),
                   jax.ShapeDtypeStruct((E // BE * 8, 128), _f32)],
        compiler_params=pltpu.CompilerParams(
            dimension_semantics=("parallel",)),
    )(gr2, gc2, ea1, e2c, e2b1, e2w2, e2b2, fw1, fb1)


def _run_final(z, av, cv, fw2, fb2):
    grid = (E // BE,)
    blk = lambda r, c: pl.BlockSpec((r, c), lambda i: (i, 0))
    return pl.pallas_call(
        _final_body,
        grid=grid,
        in_specs=[blk(BE, 128), _w_spec((1, 128)), _w_spec((1, 128)),
                  _w_spec((128, 3)), _w_spec((1, 3))],
        out_specs=blk(BE, 3),
        out_shape=jax.ShapeDtypeStruct((E, 3), _f32),
        compiler_params=pltpu.CompilerParams(
            dimension_semantics=("parallel",)),
    )(z, av, cv, fw2, fb2)


def _sc_mesh():
    return plsc.VectorSubcoreMesh(core_axis_name="c", subcore_axis_name="s")


def _gather_pair(tbl, idx_hbm, out_hbm, base, idx_v, rows_v, sem, k):
    pltpu.sync_copy(idx_hbm.at[pl.ds(base, k * CH)], idx_v.at[pl.ds(0, k * CH)])
    hs = [pltpu.async_copy(tbl.at[idx_v.at[pl.ds(j * CH, CH)]],
                           rows_v.at[pl.ds(j * CH, CH)], sem)
          for j in range(k)]
    for h in hs:
        h.wait()
    pltpu.sync_copy(rows_v.at[pl.ds(0, k * CH)], out_hbm.at[pl.ds(base, k * CH)])


def _gather_chunks(wid, tbl, idx_hbm, out_hbm, nchunks, idx_v, rows_v, sem):
    nquads = nchunks // 4
    iters = (nquads + NW - 1) // NW

    @pl.loop(0, iters)
    def _(i):
        quad = i * NW + wid

        @pl.when(quad < nquads)
        def _():
            _gather_pair(tbl, idx_hbm, out_hbm, quad * (4 * CH),
                         idx_v, rows_v, sem, 4)

    rem = nchunks - nquads * 4
    if rem:
        @pl.when(wid == 0)
        def _():
            _gather_pair(tbl, idx_hbm, out_hbm, nquads * (4 * CH),
                         idx_v, rows_v, sem, rem)


_SC_SCRATCH = [pltpu.VMEM((4 * CH,), jnp.int32),
               pltpu.VMEM((4 * CH, 128), _f32),
               pltpu.SemaphoreType.DMA]


def _sc_gather3(p1, p2, p3, rowE, colE, rowN):
    @functools.partial(
        pl.kernel,
        mesh=_sc_mesh(),
        out_type=(jax.ShapeDtypeStruct((E, 128), _f32),
                  jax.ShapeDtypeStruct((E, 128), _f32),
                  jax.ShapeDtypeStruct((NP, 128), _f32)),
        scratch_types=list(_SC_SCRATCH),
    )
    def k(p1_h, p2_h, p3_h, row_h, col_h, rown_h, gr_h, gc_h, g3_h,
          idx_v, rows_v, sem):
        wid = lax.axis_index("s") * 2 + lax.axis_index("c")
        _gather_chunks(wid, p1_h, row_h, gr_h, E // CH, idx_v, rows_v, sem)
        _gather_chunks(wid, p2_h, col_h, gc_h, E // CH, idx_v, rows_v, sem)
        _gather_chunks(wid, p3_h, rown_h, g3_h, NP // CH, idx_v, rows_v, sem)

    return k(p1, p2, p3, rowE, colE, rowN)


def _sc_gather2(q1, q2, rowE, colE):
    @functools.partial(
        pl.kernel,
        mesh=_sc_mesh(),
        out_type=(jax.ShapeDtypeStruct((E, 128), _f32),
                  jax.ShapeDtypeStruct((E, 128), _f32)),
        scratch_types=list(_SC_SCRATCH),
    )
    def k(q1_h, q2_h, row_h, col_h, gr_h, gc_h, idx_v, rows_v, sem):
        wid = lax.axis_index("s") * 2 + lax.axis_index("c")
        _gather_chunks(wid, q1_h, row_h, gr_h, E // CH, idx_v, rows_v, sem)
        _gather_chunks(wid, q2_h, col_h, gc_h, E // CH, idx_v, rows_v, sem)

    return k(q1, q2, rowE, colE)


def kernel(x, edge_index, edge_attr, batch, node_type, emb, conv_w, conv_b,
           fuse_w, fuse_b, e1_w1, e1_b1, e1_w2, e1_b2, n1_w1, n1_b1, n1_w2,
           n1_b2, e2_w1, e2_b1, e2_w2, e2_b2, n2_w1, n2_b1, n2_w2, n2_b2,
           f_w1, f_b1, bn_g, bn_b, f_w2, f_b2):
    L = 5
    _b = lambda v: jax.lax.reduce_precision(v, 8, 7)
    cwb = _b(conv_w)
    embb = _b(emb)
    w0 = cwb[:, :, 0]; w1 = cwb[:, :, 1]; w2 = cwb[:, :, 2]
    ws = w0 + w1 + w2
    M = jnp.concatenate([ws[:, :4].T, -w2[:, :4].T, -w0[:, :4].T], axis=0) / L
    s_pe = embb.sum(0)
    const = (s_pe @ ws[:, 4:].T - embb[4] @ w0[:, 4:].T - embb[0] @ w2[:, 4:].T) / L + conv_b
    S = np.zeros((20, 12), np.float32)
    for i in range(4):
        for l in range(5):
            S[l * 4 + i, i] = 1.0
        S[0 * 4 + i, 4 + i] = 1.0
        S[4 * 4 + i, 8 + i] = 1.0
    WSM = jnp.asarray(S) @ M
    cb = const.reshape(1, 128)
    Xn = jax.lax.reduce_precision(x.reshape(N, 20), 8, 7)

    rowE = edge_index[0]
    colE = edge_index[1]
    rowN = jnp.concatenate([edge_index[0, :N], jnp.zeros((NP - N,), jnp.int32)])

    r1 = lambda v: v.reshape(1, -1)

    p1, p2, p3 = _run_node_enc(Xn, node_type, WSM, cb, fuse_w[:128],
                               fuse_w[128:129], r1(fuse_b), e1_w1[:128],
                               e1_w1[128:256], n1_w1[:128])
    gr, gc, g3p = _sc_gather3(p1, p2, p3, rowE, colE, rowN)
    ea1 = _run_edge1(gr, gc, edge_attr, e1_w1[256:260], r1(e1_b1),
                     e1_w2, r1(e1_b2))
    q1, q2 = _run_node1(g3p, ea1, n1_w1[128:256], r1(n1_b1), n1_w2, r1(n1_b2),
                        e2_w1[:128], e2_w1[128:256])
    gr2, gc2 = _sc_gather2(q1, q2, rowE, colE)
    z, st = _run_edge2(gr2, gc2, ea1, e2_w1[256:384], r1(e2_b1), e2_w2,
                       r1(e2_b2), f_w1, r1(f_b1))
    stb = st.reshape(E // BE, 8, 128)
    mu = stb[:, 0].sum(0) / E
    var = stb[:, 1].sum(0) / E - mu * mu
    a = bn_g / jnp.sqrt(var + 1e-5)
    c = bn_b - mu * a
    return _run_final(z, r1(a), r1(c), f_w2, r1(f_b2))
